# Initial kernel scaffold; baseline (speedup 1.0000x reference)
import jax
import jax.numpy as jnp
from jax.experimental import pallas as pl


def kernel(feature, edge_index, Wc1, bc1, Wc2, bc2, Wc3, bc3, Ws, bs, M1w, M1b, M2w, M2b, M3w, M3b, Nw1, Nb1, Nw2, Nb2, Nw3, Nb3):
    raise NotImplementedError("write your pallas kernel here")


# speedup vs baseline: 5.6617x; 5.6617x over previous
import functools
import jax
import jax.numpy as jnp
from jax import lax
from jax.experimental import pallas as pl
from jax.experimental.pallas import tpu as pltpu
from jax.experimental.pallas import tpu_sc as plsc

N = 10000
E = 320000
D = 128
B = 10
NPER = 1000
KTOP = 500
NC = 10

NPAD = 10240
CH = 128
NCHW = 80
NW = 32
EPAD = NW * NCHW * CH
RPT = NPAD // 16


def _make_edge_pass(dw, interpret=False):
  mesh = plsc.VectorSubcoreMesh(core_axis_name="c", subcore_axis_name="s")

  @functools.partial(
      pl.kernel,
      out_type=jax.ShapeDtypeStruct((2, NPAD, dw), jnp.float32),
      mesh=mesh,
      interpret=interpret,
      compiler_params=pltpu.CompilerParams(use_tc_tiling_on_sc=(dw % 128 == 0)),
      scratch_types=[
          pltpu.VMEM((NCHW, CH), jnp.int32),
          pltpu.VMEM((NCHW, CH), jnp.int32),
          pltpu.VMEM((CH, dw), jnp.float32),
          pltpu.VMEM_SHARED((NPAD, dw), jnp.float32),
          pltpu.SemaphoreType.DMA,
      ],
  )
  def edge_pass(table, srcr, dstr, zrows, out, src_v, dst_v, rows_v, acc, sem):
    c = lax.axis_index("c")
    s = lax.axis_index("s")
    pltpu.sync_copy(srcr.at[c, s], src_v)
    pltpu.sync_copy(dstr.at[c, s], dst_v)
    pltpu.sync_copy(zrows, acc.at[pl.ds(s * RPT, RPT)])
    plsc.subcore_barrier()

    def body(j, carry):
      pltpu.async_copy(table.at[src_v.at[j]], rows_v, sem).wait()
      pltpu.sync_copy(rows_v, acc.at[dst_v.at[j]], add=True)
      return carry

    lax.fori_loop(0, NCHW, body, 0)
    plsc.subcore_barrier()
    pltpu.sync_copy(acc.at[pl.ds(s * RPT, RPT)],
                    out.at[c, pl.ds(s * RPT, RPT)])

  return edge_pass


def _make_deg_pass(interpret=False):
  mesh = plsc.VectorSubcoreMesh(core_axis_name="c", subcore_axis_name="s")

  @functools.partial(
      pl.kernel,
      out_type=(jax.ShapeDtypeStruct((2, NPAD, 16), jnp.float32),
                jax.ShapeDtypeStruct((2, NPAD, 16), jnp.float32)),
      mesh=mesh,
      interpret=interpret,
      compiler_params=pltpu.CompilerParams(use_tc_tiling_on_sc=False),
      scratch_types=[
          pltpu.VMEM((NCHW, CH), jnp.int32),
          pltpu.VMEM((NCHW, CH), jnp.int32),
          pltpu.VMEM((CH, 16), jnp.float32),
          pltpu.VMEM_SHARED((NPAD, 16), jnp.float32),
          pltpu.VMEM_SHARED((NPAD, 16), jnp.float32),
      ],
  )
  def deg_pass(ones_h, srcr, dstr, zrows, out_in, out_out,
               src_v, dst_v, ones_v, acc_in, acc_out):
    c = lax.axis_index("c")
    s = lax.axis_index("s")
    pltpu.sync_copy(srcr.at[c, s], src_v)
    pltpu.sync_copy(dstr.at[c, s], dst_v)
    pltpu.sync_copy(ones_h, ones_v)
    pltpu.sync_copy(zrows, acc_in.at[pl.ds(s * RPT, RPT)])
    pltpu.sync_copy(zrows, acc_out.at[pl.ds(s * RPT, RPT)])
    plsc.subcore_barrier()

    def body(j, carry):
      pltpu.sync_copy(ones_v, acc_in.at[dst_v.at[j]], add=True)
      pltpu.sync_copy(ones_v, acc_out.at[src_v.at[j]], add=True)
      return carry

    lax.fori_loop(0, NCHW, body, 0)
    plsc.subcore_barrier()
    pltpu.sync_copy(acc_in.at[pl.ds(s * RPT, RPT)],
                    out_in.at[c, pl.ds(s * RPT, RPT)])
    pltpu.sync_copy(acc_out.at[pl.ds(s * RPT, RPT)],
                    out_out.at[c, pl.ds(s * RPT, RPT)])

  return deg_pass


_BLK = 1024


def _prep_body(x_ref, din_ref, dout_ref, w_ref, scaled_ref, sn_ref, dn_ref):
  din = din_ref[0] + din_ref[1]
  dout = dout_ref[0] + dout_ref[1]
  dn = lax.rsqrt(jnp.maximum(din[:, 0:1], 1.0))
  sn = lax.rsqrt(jnp.maximum(dout[:, 0:1], 1.0))
  xw = jnp.dot(x_ref[...], w_ref[...], preferred_element_type=jnp.float32)
  scaled_ref[...] = xw * sn
  sn_ref[...] = sn
  dn_ref[...] = dn


def _layer_body(x_ref, p_ref, dn_ref, sn_ref, b_ref, w_ref,
                out_ref, mean_ref, max_ref, scaled_ref):
  agg = (p_ref[0] + p_ref[1]) * dn_ref[...] + b_ref[...]
  out = x_ref[...] + jnp.maximum(agg, 0.0)
  out_ref[...] = out
  mean_ref[...] = jnp.broadcast_to(
      jnp.sum(out, axis=0, keepdims=True)[None] * (1.0 / NPER), (1, 8, D))
  max_ref[...] = jnp.broadcast_to(
      jnp.max(out, axis=0, keepdims=True)[None], (1, 8, D))
  xw = jnp.dot(out, w_ref[...], preferred_element_type=jnp.float32)
  scaled_ref[...] = xw * sn_ref[...]


def _layer3_body(x_ref, p_ref, dn_ref, sn_ref, b_ref, ws_ref,
                 nw1_ref, nb1_ref, nw2_ref, nb2_ref, nw3_ref, nb3_ref,
                 out_ref, sin_ref, npred_ref):
  agg = (p_ref[0] + p_ref[1]) * dn_ref[...] + b_ref[...]
  out = x_ref[...] + jnp.maximum(agg, 0.0)
  out_ref[...] = out
  sxw = jnp.dot(out, ws_ref[...], preferred_element_type=jnp.float32)
  sin_ref[...] = jnp.broadcast_to(sxw * sn_ref[...], (NPER, 16))
  h = jnp.maximum(
      jnp.dot(out, nw1_ref[...], preferred_element_type=jnp.float32)
      + nb1_ref[...], 0.0)
  h = jnp.maximum(
      jnp.dot(h, nw2_ref[...], preferred_element_type=jnp.float32)
      + nb2_ref[...], 0.0)
  npred_ref[...] = (jnp.dot(h, nw3_ref[...], preferred_element_type=jnp.float32)
                    + nb3_ref[...])


def _final_body(out3_ref, sp_ref, dn_ref, bs_ref,
                fdm_ref, fdx_ref, fcm_ref, fcx_ref, fum_ref, fux_ref,
                score_ref, srow_ref):
  sc = (sp_ref[0] + sp_ref[1])[:, 0:1] * dn_ref[...] + bs_ref[...]
  score_ref[...] = sc
  f3 = out3_ref[...] * jnp.tanh(sc)
  srow_ref[...] = jnp.full((1, 1024), jnp.nan, jnp.float32)
  srow_ref[:, :NPER] = sc.reshape(1, NPER)
  ii = lax.broadcasted_iota(jnp.int32, (NPER, 128), 0)
  jjb = lax.broadcasted_iota(jnp.int32, (NPER, 128), 1)

  def rank_chunk(j, rank):
    chunk = srow_ref[:, pl.ds(j * 128, 128)]
    jj = jjb + j * 128
    gt = chunk > sc
    tie = (chunk == sc) & (jj < ii)
    return rank + jnp.sum(gt.astype(jnp.float32) + tie.astype(jnp.float32),
                          axis=1, keepdims=True)

  rank = lax.fori_loop(0, 8, rank_chunk, jnp.zeros((NPER, 1), jnp.float32))
  mask = rank < float(KTOP)
  neg = jnp.float32(-jnp.inf)
  fd = jnp.where(mask, f3, 0.0)
  fc = jnp.where(mask, 0.0, f3)
  bc8 = lambda r: jnp.broadcast_to(r[None], (1, 8, D))
  fdm_ref[...] = bc8(jnp.sum(fd, axis=0, keepdims=True) * (1.0 / KTOP))
  fcm_ref[...] = bc8(jnp.sum(fc, axis=0, keepdims=True) * (1.0 / (NPER - KTOP)))
  fdx_ref[...] = bc8(jnp.max(jnp.where(mask, f3, neg), axis=0, keepdims=True))
  fcx_ref[...] = bc8(jnp.max(jnp.where(mask, neg, f3), axis=0, keepdims=True))
  fum_ref[...] = bc8(jnp.sum(f3, axis=0, keepdims=True) * (1.0 / NPER))
  fux_ref[...] = bc8(jnp.max(f3, axis=0, keepdims=True))


def _small_body(hg1m_ref, hg1x_ref, hg2m_ref, hg2x_ref,
                fdm_ref, fdx_ref, fcm_ref, fcx_ref, fum_ref, fux_ref,
                m1w_ref, m1b_ref, m2w_ref, m2b_ref, m3w_ref, m3b_ref,
                s_ref, scom_ref, sfull_ref):
  h12m = hg1m_ref[:, 0, :] + hg2m_ref[:, 0, :]
  h12x = hg1x_ref[:, 0, :] + hg2x_ref[:, 0, :]
  fdm, fdx = fdm_ref[:, 0, :], fdx_ref[:, 0, :]
  fcm, fcx = fcm_ref[:, 0, :], fcx_ref[:, 0, :]
  fum, fux = fum_ref[:, 0, :], fux_ref[:, 0, :]
  hg = jnp.concatenate([h12m + fdm, h12x + fdx], axis=1)
  hg_com = jnp.concatenate([fcm, fcx], axis=1)
  hg_full = jnp.concatenate([h12m + fum, h12x + fux], axis=1)
  x = jnp.concatenate([hg, hg_com, hg_full], axis=0)
  h = jnp.maximum(
      jnp.dot(x, m1w_ref[...], preferred_element_type=jnp.float32)
      + m1b_ref[...], 0.0)
  h = jnp.maximum(
      jnp.dot(h, m2w_ref[...], preferred_element_type=jnp.float32)
      + m2b_ref[...], 0.0)
  y = jnp.dot(h, m3w_ref[...], preferred_element_type=jnp.float32) + m3b_ref[...]
  s_ref[...] = y[0:B]
  scom_ref[...] = y[B:2 * B]
  sfull_ref[...] = y[2 * B:3 * B]


def _softmax_body(x_ref, o_ref):
  x = x_ref[...]
  m = jnp.max(x)
  e = jnp.exp(x - m)
  o_ref[...] = e / jnp.sum(e)


def kernel(feature, edge_index, Wc1, bc1, Wc2, bc2, Wc3, bc3, Ws, bs,
           M1w, M1b, M2w, M2b, M3w, M3b, Nw1, Nb1, Nw2, Nb2, Nw3, Nb3):
  f32 = jnp.float32
  interp = False

  xpad = jnp.pad(feature, ((0, NPAD - N), (0, 0)))
  src = edge_index[0]
  dst = edge_index[1]
  padlen = EPAD - E
  padv = jnp.full((padlen,), N, jnp.int32)
  srcp = jnp.concatenate([src, padv]).reshape(2, 16, NCHW, CH)
  dstp = jnp.concatenate([dst, padv]).reshape(2, 16, NCHW, CH)
  z16 = jnp.zeros((RPT, 16), f32)
  z128 = jnp.zeros((RPT, D), f32)
  ones16 = jnp.ones((CH, 16), f32)

  edge128 = _make_edge_pass(D, interp)
  edge16 = _make_edge_pass(16, interp)
  deg_pass = _make_deg_pass(interp)

  din_p, dout_p = deg_pass(ones16, srcp, dstp, z16)

  gl = NPAD // _BLK
  io128 = pl.BlockSpec((_BLK, D), lambda i: (i, 0))
  io1 = pl.BlockSpec((_BLK, 1), lambda i: (i, 0))
  iod = pl.BlockSpec((2, _BLK, 16), lambda i: (0, i, 0))
  wfull = pl.BlockSpec((D, D), lambda i: (0, 0))

  scaled1, sn, dn = pl.pallas_call(
      _prep_body,
      grid=(gl,),
      in_specs=[io128, iod, iod, wfull],
      out_specs=[io128, io1, io1],
      out_shape=[jax.ShapeDtypeStruct((NPAD, D), f32),
                 jax.ShapeDtypeStruct((NPAD, 1), f32),
                 jax.ShapeDtypeStruct((NPAD, 1), f32)],
      interpret=interp,
  )(xpad, din_p, dout_p, Wc1)

  gb128 = pl.BlockSpec((NPER, D), lambda i: (i, 0))
  gb1 = pl.BlockSpec((NPER, 1), lambda i: (i, 0))
  gbp = pl.BlockSpec((2, NPER, D), lambda i: (0, i, 0))
  row128 = pl.BlockSpec((1, D), lambda i: (0, 0))
  hgrow = pl.BlockSpec((1, 8, D), lambda i: (i, 0, 0))
  hgshape = jax.ShapeDtypeStruct((B, 8, D), f32)

  layer_call = pl.pallas_call(
      _layer_body,
      grid=(B,),
      in_specs=[gb128, gbp, gb1, gb1, row128, wfull],
      out_specs=[gb128, hgrow, hgrow, gb128],
      out_shape=[jax.ShapeDtypeStruct((NPAD, D), f32),
                 hgshape, hgshape,
                 jax.ShapeDtypeStruct((NPAD, D), f32)],
      interpret=interp,
  )

  p1 = edge128(scaled1, srcp, dstp, z128)
  out1, hg1m, hg1x, scaled2 = layer_call(
      xpad, p1, dn, sn, bc1.reshape(1, D), Wc2)

  p2 = edge128(scaled2, srcp, dstp, z128)
  out2, hg2m, hg2x, scaled3 = layer_call(
      out1, p2, dn, sn, bc2.reshape(1, D), Wc3)

  p3 = edge128(scaled3, srcp, dstp, z128)
  wsfull = pl.BlockSpec((D, 1), lambda i: (0, 0))
  out3, score_in, npred = pl.pallas_call(
      _layer3_body,
      grid=(B,),
      in_specs=[gb128, gbp, gb1, gb1, row128, wsfull,
                pl.BlockSpec((D, D // 2), lambda i: (0, 0)),
                pl.BlockSpec((1, D // 2), lambda i: (0, 0)),
                pl.BlockSpec((D // 2, D // 4), lambda i: (0, 0)),
                pl.BlockSpec((1, D // 4), lambda i: (0, 0)),
                pl.BlockSpec((D // 4, NC), lambda i: (0, 0)),
                pl.BlockSpec((1, NC), lambda i: (0, 0))],
      out_specs=[gb128,
                 pl.BlockSpec((NPER, 16), lambda i: (i, 0)),
                 pl.BlockSpec((NPER, NC), lambda i: (i, 0))],
      out_shape=[jax.ShapeDtypeStruct((NPAD, D), f32),
                 jax.ShapeDtypeStruct((NPAD, 16), f32),
                 jax.ShapeDtypeStruct((NPAD, NC), f32)],
      interpret=interp,
  )(out2, p3, dn, sn, bc3.reshape(1, D), Ws,
    Nw1, Nb1.reshape(1, D // 2), Nw2, Nb2.reshape(1, D // 4),
    Nw3, Nb3.reshape(1, NC))

  p4 = edge16(score_in, srcp, dstp, z16)

  gbp16 = pl.BlockSpec((2, NPER, 16), lambda i: (0, i, 0))
  one1 = pl.BlockSpec((1, 1), lambda i: (0, 0))
  fdm, fdx, fcm, fcx, fum, fux, score = pl.pallas_call(
      _final_body,
      grid=(B,),
      in_specs=[gb128, gbp16, gb1, one1],
      out_specs=[hgrow, hgrow, hgrow, hgrow, hgrow, hgrow, gb1],
      out_shape=[hgshape] * 6 + [jax.ShapeDtypeStruct((NPAD, 1), f32)],
      scratch_shapes=[pltpu.VMEM((1, 1024), jnp.float32)],
      interpret=interp,
  )(out3, p4, dn, bs.reshape(1, 1))

  full = lambda r, c: pl.BlockSpec((r, c), lambda: (0, 0))
  full3 = pl.BlockSpec((B, 8, D), lambda: (0, 0, 0))
  scores, scores_com, scores_full = pl.pallas_call(
      _small_body,
      in_specs=[full3] * 10
      + [full(2 * D, D), full(1, D), full(D, D // 2), full(1, D // 2),
         full(D // 2, NC), full(1, NC)],
      out_specs=[full(B, NC)] * 3,
      out_shape=[jax.ShapeDtypeStruct((B, NC), f32)] * 3,
      interpret=interp,
  )(hg1m, hg1x, hg2m, hg2x, fdm, fdx, fcm, fcx, fum, fux,
    M1w, M1b.reshape(1, D), M2w, M2b.reshape(1, D // 2),
    M3w, M3b.reshape(1, NC))

  sflat = score[:N, 0].reshape(8, N // 8)
  node_score = pl.pallas_call(
      _softmax_body,
      in_specs=[full(8, N // 8)],
      out_specs=full(8, N // 8),
      out_shape=jax.ShapeDtypeStruct((8, N // 8), f32),
  )(sflat).reshape(N)

  return (scores, scores_com, scores_full, npred[:N], node_score)
